# baseline (speedup 1.0000x reference)
import functools

import jax
import jax.numpy as jnp
from jax import lax
from jax.experimental import pallas as pl
from jax.experimental.pallas import tpu as pltpu

ALPHA = 0.6
B = 16384
C = 1000

RB = 512
JB = 512
KC = 2048


def _ce_kernel(x_ref, y_ref, out_ref):
    i = pl.program_id(0)
    x = x_ref[...]
    y = y_ref[0, 0, :]
    col = lax.broadcasted_iota(jnp.int32, (RB, C), 1)
    m = jnp.max(x, axis=1, keepdims=True)
    lse = jnp.log(jnp.sum(jnp.exp(x - m), axis=1, keepdims=True)) + m
    tgt = jnp.sum(jnp.where(col == y[:, None], x, 0.0), axis=1, keepdims=True)
    part = jnp.sum(lse - tgt).reshape(1, 1)

    @pl.when(i == 0)
    def _():
        out_ref[...] = jnp.zeros((1, 1), jnp.float32)

    out_ref[...] += part


_NT = (((1,), (1,)), ((), ()))


def _cox_rank_kernel(durj_ref, dur_row_ref, cox_row_ref, risk_ref, rank_ref):
    i = pl.program_id(0)
    j0 = i * JB
    dj = durj_ref[...]
    jid = j0 + lax.broadcasted_iota(jnp.int32, (JB, 1), 0)

    def body(kk, acc):
        k0 = kk * KC
        dk = dur_row_ref[:, pl.ds(k0, KC)]
        kid = k0 + lax.broadcasted_iota(jnp.int32, (1, KC), 1)
        m = (dk > dj) | ((dk == dj) & (kid <= jid))
        mf = m.astype(jnp.float32)
        hz = jnp.exp(cox_row_ref[:, pl.ds(k0, KC)])
        rhs = jnp.concatenate([hz, jnp.ones((1, KC), jnp.float32)], axis=0)
        return acc + lax.dot_general(mf, rhs, _NT,
                                     preferred_element_type=jnp.float32)

    acc = lax.fori_loop(0, B // KC, body, jnp.zeros((JB, 2), jnp.float32))
    risk_ref[...] = acc[:, 0:1]
    rank_ref[...] = acc[:, 1:2] - 1.0


def _cox_mask_kernel(rankj_ref, ev_row_ref, coxj_ref, riskj_ref, out_ref):
    i = pl.program_id(0)
    rj = rankj_ref[...]

    def body(kk, acc):
        k0 = kk * KC
        kidf = (k0 + lax.broadcasted_iota(jnp.int32, (1, KC), 1)).astype(jnp.float32)
        onehot = (rj == kidf).astype(jnp.float32)
        ev = ev_row_ref[:, pl.ds(k0, KC)].astype(jnp.float32)
        return acc + lax.dot_general(onehot, ev, _NT,
                                     preferred_element_type=jnp.float32)

    w = lax.fori_loop(0, B // KC, body, jnp.zeros((JB, 1), jnp.float32))
    diff = coxj_ref[...] - jnp.log(riskj_ref[...] + 1e-8)
    part = jnp.sum(w * diff).reshape(1, 1)

    @pl.when(i == 0)
    def _():
        cnt = jnp.sum(ev_row_ref[...].astype(jnp.float32)).reshape(1, 1)
        out_ref[...] = jnp.concatenate([jnp.zeros((1, 1), jnp.float32), cnt],
                                       axis=1)

    out_ref[...] += jnp.concatenate([part, jnp.zeros((1, 1), jnp.float32)],
                                    axis=1)


def kernel(cls_out, y_cls, cox_out, durations, events):
    y3d = y_cls.astype(jnp.int32).reshape(B // RB, 1, RB)
    ce_sum = pl.pallas_call(
        _ce_kernel,
        grid=(B // RB,),
        in_specs=[
            pl.BlockSpec((RB, C), lambda i: (i, 0)),
            pl.BlockSpec((1, 1, RB), lambda i: (i, 0, 0)),
        ],
        out_specs=pl.BlockSpec((1, 1), lambda i: (0, 0)),
        out_shape=jax.ShapeDtypeStruct((1, 1), jnp.float32),
    )(cls_out, y3d)

    dur2d = durations.reshape(B, 1)
    cox2d = cox_out.reshape(B, 1)
    dur_row = durations.reshape(1, B)
    cox_row = cox_out.reshape(1, B)
    ev_row = events.astype(jnp.int32).reshape(1, B)

    risk, rank = pl.pallas_call(
        _cox_rank_kernel,
        grid=(B // JB,),
        in_specs=[
            pl.BlockSpec((JB, 1), lambda i: (i, 0)),
            pl.BlockSpec((1, B), lambda i: (0, 0)),
            pl.BlockSpec((1, B), lambda i: (0, 0)),
        ],
        out_specs=[
            pl.BlockSpec((JB, 1), lambda i: (i, 0)),
            pl.BlockSpec((JB, 1), lambda i: (i, 0)),
        ],
        out_shape=[
            jax.ShapeDtypeStruct((B, 1), jnp.float32),
            jax.ShapeDtypeStruct((B, 1), jnp.float32),
        ],
    )(dur2d, dur_row, cox_row)

    cox_part = pl.pallas_call(
        _cox_mask_kernel,
        grid=(B // JB,),
        in_specs=[
            pl.BlockSpec((JB, 1), lambda i: (i, 0)),
            pl.BlockSpec((1, B), lambda i: (0, 0)),
            pl.BlockSpec((JB, 1), lambda i: (i, 0)),
            pl.BlockSpec((JB, 1), lambda i: (i, 0)),
        ],
        out_specs=pl.BlockSpec((1, 2), lambda i: (0, 0)),
        out_shape=jax.ShapeDtypeStruct((1, 2), jnp.float32),
    )(rank, ev_row, cox2d, risk)

    loss_cls = ce_sum[0, 0] / jnp.float32(B)
    masked_sum = cox_part[0, 0]
    cnt = cox_part[0, 1]
    loss_cox = jnp.where(cnt == 0.0, jnp.float32(0.0),
                         -(masked_sum / jnp.maximum(cnt, 1.0)))
    total = ALPHA * loss_cls + (1.0 - ALPHA) * loss_cox
    return (total, loss_cls, loss_cox)


# speedup vs baseline: 5.4089x; 5.4089x over previous
import functools

import jax
import jax.numpy as jnp
from jax import lax
from jax.experimental import pallas as pl
from jax.experimental.pallas import tpu as pltpu
from jax.experimental.pallas import tpu_sc as plsc

ALPHA = 0.6
B = 16384
C = 1000

RB = 512

NW = 16
CH = B // NW
CL = CH // 16
NPASS = 4


def _ce_kernel(x_ref, y_ref, out_ref):
    i = pl.program_id(0)
    x = x_ref[...]
    y = y_ref[0, 0, :]
    col = lax.broadcasted_iota(jnp.int32, (RB, C), 1)
    m = jnp.max(x, axis=1, keepdims=True)
    lse = jnp.log(jnp.sum(jnp.exp(x - m), axis=1, keepdims=True)) + m
    tgt = jnp.sum(jnp.where(col == y[:, None], x, 0.0), axis=1, keepdims=True)
    part = jnp.sum(lse - tgt).reshape(1, 1)

    @pl.when(i == 0)
    def _():
        out_ref[...] = jnp.zeros((1, 1), jnp.float32)

    out_ref[...] += part


def _vlog(x):
    bits = plsc.bitcast(x, jnp.int32)
    e = lax.shift_right_logical(bits, 23) - 127
    m = plsc.bitcast((bits & 0x7FFFFF) | 0x3F800000, jnp.float32)
    big = m > 1.4142135
    m = jnp.where(big, m * 0.5, m)
    ef = (e + big.astype(jnp.int32)).astype(jnp.float32)
    s = (m - 1.0) / (m + 1.0)
    s2 = s * s
    p = 2.0 + s2 * (0.6666666666666667 + s2 * (0.4 + s2 * (0.2857142857142857
                                                           + s2 * 0.2222222222222222)))
    return ef * 0.6931471805599453 + s * p


def _cox_sc_body(dur_hbm, cox_hbm, ev_hbm, out_hbm,
                 ska, skb, sva, svb, sghw, stot, spm, spc,
                 keych, valch, posbuf, hist3, counts3, ghwloc, histw, epw,
                 riskloc, evch, redloc, vbuf):
    wid = lax.axis_index("s") + lax.axis_index("c") * 0
    base = wid * CH
    lane = lax.iota(jnp.int32, 16)
    z16 = jnp.zeros((16,), jnp.int32)
    ones16 = jnp.ones((16,), jnp.int32)

    pltpu.sync_copy(dur_hbm.at[pl.ds(base, CH)], riskloc)

    def kinit(t, _):
        bits = plsc.bitcast(riskloc[pl.ds(t * 16, 16)], jnp.int32)
        sign = lax.shift_right_arithmetic(bits, 31)
        ub = bits ^ (sign | jnp.int32(-2147483648))
        keych[pl.ds(t * 16, 16)] = ~ub
        return 0

    lax.fori_loop(0, CL, kinit, 0)
    pltpu.sync_copy(cox_hbm.at[pl.ds(base, CH)], valch)

    bufs = ((ska, sva), (skb, svb))
    for p in range(NPASS):
        cur_k, cur_v = bufs[p % 2]
        nxt_k, nxt_v = bufs[(p + 1) % 2]
        if p > 0:
            pltpu.sync_copy(cur_k.at[pl.ds(base, CH)], keych)
            pltpu.sync_copy(cur_v.at[pl.ds(base, CH)], valch)
        shift = 8 * p

        def hzero(t, _):
            hist3[pl.ds(t * 16, 16)] = z16
            return 0

        lax.fori_loop(0, 256, hzero, 0)

        def hbody(i, _):
            idx = lane * CL + i
            k = plsc.load_gather(keych, [idx])
            d = lax.shift_right_logical(k, shift) & 255
            plsc.addupdate_scatter(hist3, [lane * 256 + d], ones16)
            return 0

        lax.fori_loop(0, CL, hbody, 0)

        def wsum(g, _):
            def lsum(l, acc):
                return acc + hist3[pl.ds(l * 256 + g * 16, 16)]

            histw[pl.ds(g * 16, 16)] = lax.fori_loop(0, 16, lsum, z16)
            return 0

        lax.fori_loop(0, 16, wsum, 0)
        pltpu.sync_copy(histw, sghw.at[pl.ds(wid * 256, 256)])
        plsc.subcore_barrier()
        pltpu.sync_copy(sghw, ghwloc)

        def gbody(g, carry):
            def wacc(w2, tp):
                t_acc, p_acc = tp
                row = ghwloc[pl.ds(w2 * 256 + g * 16, 16)]
                return (t_acc + row,
                        p_acc + row * (w2 < wid).astype(jnp.int32))

            t_tot, p_pre = lax.fori_loop(0, NW, wacc, (z16, z16))
            ex = plsc.cumsum(t_tot) - t_tot
            epw[pl.ds(g * 16, 16)] = ex + p_pre + carry
            return carry + jnp.sum(t_tot)

        lax.fori_loop(0, 16, gbody, jnp.int32(0))

        def cinit(g, _):
            e_slice = epw[pl.ds(g * 16, 16)]

            def linit(l, run):
                counts3[pl.ds(l * 256 + g * 16, 16)] = e_slice + run
                return run + hist3[pl.ds(l * 256 + g * 16, 16)]

            lax.fori_loop(0, 16, linit, z16)
            return 0

        lax.fori_loop(0, 16, cinit, 0)

        def pbody(i, _):
            idx = lane * CL + i
            k = plsc.load_gather(keych, [idx])
            d = lax.shift_right_logical(k, shift) & 255
            fl = lane * 256 + d
            c = plsc.load_gather(counts3, [fl])
            plsc.addupdate_scatter(counts3, [fl], ones16)
            plsc.store_scatter(posbuf, [idx], c)
            return 0

        lax.fori_loop(0, CL, pbody, 0)
        pltpu.sync_copy(keych, nxt_k.at[posbuf])
        pltpu.sync_copy(valch, nxt_v.at[posbuf])
        plsc.subcore_barrier()

    fin_v = bufs[NPASS % 2][1]
    pltpu.sync_copy(fin_v.at[pl.ds(base, CH)], valch)
    pltpu.sync_copy(ev_hbm.at[pl.ds(base, CH)], evch)

    def cumb(t, carry):
        h = jnp.exp(valch[pl.ds(t * 16, 16)])
        riskloc[pl.ds(t * 16, 16)] = plsc.cumsum(h) + carry
        return carry + jnp.sum(h)

    tot = lax.fori_loop(0, CL, cumb, jnp.float32(0.0))
    vbuf[...] = jnp.zeros((16,), jnp.float32) + tot
    pltpu.sync_copy(vbuf, stot.at[pl.ds(wid * 16, 16)])
    plsc.subcore_barrier()
    pltpu.sync_copy(stot, redloc)

    def prb(w2, acc):
        return acc + redloc[pl.ds(w2 * 16, 16)] * (w2 < wid).astype(jnp.float32)

    prevec = lax.fori_loop(0, NW, prb, jnp.zeros((16,), jnp.float32))

    def fbody(t, accs):
        am, ac = accs
        risk = riskloc[pl.ds(t * 16, 16)] + prevec
        lr = _vlog(risk + 1e-8)
        e = evch[pl.ds(t * 16, 16)].astype(jnp.float32)
        return (am + e * (valch[pl.ds(t * 16, 16)] - lr), ac + e)

    am, ac = lax.fori_loop(0, CL, fbody,
                           (jnp.zeros((16,), jnp.float32),
                            jnp.zeros((16,), jnp.float32)))
    vbuf[...] = am
    pltpu.sync_copy(vbuf, spm.at[pl.ds(wid * 16, 16)])
    vbuf[...] = ac
    pltpu.sync_copy(vbuf, spc.at[pl.ds(wid * 16, 16)])
    plsc.subcore_barrier()

    @pl.when(wid == 0)
    def _():
        pltpu.sync_copy(spm, redloc)

        def rsum(w2, acc):
            return acc + redloc[pl.ds(w2 * 16, 16)]

        zf = jnp.zeros((16,), jnp.float32)
        msv = zf + jnp.sum(lax.fori_loop(0, NW, rsum, zf))
        pltpu.sync_copy(spc, redloc)
        cntv = zf + jnp.sum(lax.fori_loop(0, NW, rsum, zf))
        vbuf[...] = jnp.where(cntv == 0.0, zf,
                              -(msv / jnp.maximum(cntv, 1.0)))
        pltpu.sync_copy(vbuf, out_hbm)


@jax.jit
def _cox_sc(durations, cox_out, ev32):
    mesh = plsc.VectorSubcoreMesh(core_axis_name="c", subcore_axis_name="s",
                                  num_cores=1)
    f = pl.kernel(
        _cox_sc_body,
        out_type=jax.ShapeDtypeStruct((16,), jnp.float32),
        mesh=mesh,
        compiler_params=pltpu.CompilerParams(needs_layout_passes=False),
        scratch_types=[
            pltpu.VMEM_SHARED((B,), jnp.int32),
            pltpu.VMEM_SHARED((B,), jnp.int32),
            pltpu.VMEM_SHARED((B,), jnp.float32),
            pltpu.VMEM_SHARED((B,), jnp.float32),
            pltpu.VMEM_SHARED((NW * 256,), jnp.int32),
            pltpu.VMEM_SHARED((NW * 16,), jnp.float32),
            pltpu.VMEM_SHARED((NW * 16,), jnp.float32),
            pltpu.VMEM_SHARED((NW * 16,), jnp.float32),
            pltpu.VMEM((CH,), jnp.int32),
            pltpu.VMEM((CH,), jnp.float32),
            pltpu.VMEM((CH,), jnp.int32),
            pltpu.VMEM((16 * 256,), jnp.int32),
            pltpu.VMEM((16 * 256,), jnp.int32),
            pltpu.VMEM((NW * 256,), jnp.int32),
            pltpu.VMEM((256,), jnp.int32),
            pltpu.VMEM((256,), jnp.int32),
            pltpu.VMEM((CH,), jnp.float32),
            pltpu.VMEM((CH,), jnp.int32),
            pltpu.VMEM((NW * 16,), jnp.float32),
            pltpu.VMEM((16,), jnp.float32),
        ],
    )
    return f(durations, cox_out, ev32)


def kernel(cls_out, y_cls, cox_out, durations, events):
    y3d = y_cls.astype(jnp.int32).reshape(B // RB, 1, RB)
    ce_sum = pl.pallas_call(
        _ce_kernel,
        grid=(B // RB,),
        in_specs=[
            pl.BlockSpec((RB, C), lambda i: (i, 0)),
            pl.BlockSpec((1, 1, RB), lambda i: (i, 0, 0)),
        ],
        out_specs=pl.BlockSpec((1, 1), lambda i: (0, 0)),
        out_shape=jax.ShapeDtypeStruct((1, 1), jnp.float32),
    )(cls_out, y3d)

    cox_vec = _cox_sc(durations, cox_out, events.astype(jnp.int32))

    loss_cls = ce_sum[0, 0] / jnp.float32(B)
    loss_cox = cox_vec[0]
    total = ALPHA * loss_cls + (1.0 - ALPHA) * loss_cox
    return (total, loss_cls, loss_cox)


# speedup vs baseline: 11.3184x; 2.0925x over previous
import functools

import jax
import jax.numpy as jnp
from jax import lax
from jax.experimental import pallas as pl
from jax.experimental.pallas import tpu as pltpu
from jax.experimental.pallas import tpu_sc as plsc

ALPHA = 0.6
B = 16384
C = 1000

CB = 1024

NW = 16
CH = B // NW
CL = CH // 16
NPASS = 4


def _ce_kernel(x_ref, y_ref, out_ref):
    i = pl.program_id(0)
    x = x_ref[...]
    y = y_ref[...]
    row = lax.broadcasted_iota(jnp.int32, (C, CB), 0)
    m = jnp.max(x, axis=0, keepdims=True)
    lse = jnp.log(jnp.sum(jnp.exp(x - m), axis=0, keepdims=True)) + m
    tgt = jnp.sum(jnp.where(row == y, x, 0.0), axis=0, keepdims=True)
    part = jnp.sum(lse - tgt).reshape(1, 1)

    @pl.when(i == 0)
    def _():
        out_ref[...] = jnp.zeros((1, 1), jnp.float32)

    out_ref[...] += part


def _vlog(x):
    bits = plsc.bitcast(x, jnp.int32)
    e = lax.shift_right_logical(bits, 23) - 127
    m = plsc.bitcast((bits & 0x7FFFFF) | 0x3F800000, jnp.float32)
    big = m > 1.4142135
    m = jnp.where(big, m * 0.5, m)
    ef = (e + big.astype(jnp.int32)).astype(jnp.float32)
    s = (m - 1.0) / (m + 1.0)
    s2 = s * s
    p = 2.0 + s2 * (0.6666666666666667 + s2 * (0.4 + s2 * (0.2857142857142857
                                                           + s2 * 0.2222222222222222)))
    return ef * 0.6931471805599453 + s * p


def _cox_sc_body(dur_hbm, cox_hbm, ev_hbm, out_hbm,
                 ska, skb, sva, svb, sghw, stot, spm, spc,
                 keych, valch, posbuf, hist3, counts3, ghwloc, histw, epw,
                 riskloc, evch, redloc, vbuf):
    wid = lax.axis_index("s") + lax.axis_index("c") * 0
    base = wid * CH
    lane = lax.iota(jnp.int32, 16)
    z16 = jnp.zeros((16,), jnp.int32)
    ones16 = jnp.ones((16,), jnp.int32)

    pltpu.sync_copy(dur_hbm.at[pl.ds(base, CH)], riskloc)

    def kinit(t, _):
        bits = plsc.bitcast(riskloc[pl.ds(t * 16, 16)], jnp.int32)
        sign = lax.shift_right_arithmetic(bits, 31)
        ub = bits ^ (sign | jnp.int32(-2147483648))
        keych[pl.ds(t * 16, 16)] = ~ub
        return 0

    lax.fori_loop(0, CL, kinit, 0)
    pltpu.sync_copy(cox_hbm.at[pl.ds(base, CH)], valch)

    bufs = ((ska, sva), (skb, svb))
    for p in range(NPASS):
        cur_k, cur_v = bufs[p % 2]
        nxt_k, nxt_v = bufs[(p + 1) % 2]
        if p > 0:
            pltpu.sync_copy(cur_k.at[pl.ds(base, CH)], keych)
            pltpu.sync_copy(cur_v.at[pl.ds(base, CH)], valch)
        shift = 8 * p

        def hzero(t, _):
            hist3[pl.ds(t * 16, 16)] = z16
            return 0

        lax.fori_loop(0, 256, hzero, 0)

        def hbody(i, _):
            idx = lane * CL + i
            k = plsc.load_gather(keych, [idx])
            d = lax.shift_right_logical(k, shift) & 255
            plsc.addupdate_scatter(hist3, [lane * 256 + d], ones16)
            return 0

        lax.fori_loop(0, CL, hbody, 0)

        def wsum(g, _):
            def lsum(l, acc):
                return acc + hist3[pl.ds(l * 256 + g * 16, 16)]

            histw[pl.ds(g * 16, 16)] = lax.fori_loop(0, 16, lsum, z16)
            return 0

        lax.fori_loop(0, 16, wsum, 0)
        pltpu.sync_copy(histw, sghw.at[pl.ds(wid * 256, 256)])
        plsc.subcore_barrier()
        pltpu.sync_copy(sghw, ghwloc)

        def gbody(g, carry):
            def wacc(w2, tp):
                t_acc, p_acc = tp
                row = ghwloc[pl.ds(w2 * 256 + g * 16, 16)]
                return (t_acc + row,
                        p_acc + row * (w2 < wid).astype(jnp.int32))

            t_tot, p_pre = lax.fori_loop(0, NW, wacc, (z16, z16))
            ex = plsc.cumsum(t_tot) - t_tot
            epw[pl.ds(g * 16, 16)] = ex + p_pre + carry
            return carry + jnp.sum(t_tot)

        lax.fori_loop(0, 16, gbody, jnp.int32(0))

        def cinit(g, _):
            e_slice = epw[pl.ds(g * 16, 16)]

            def linit(l, run):
                counts3[pl.ds(l * 256 + g * 16, 16)] = e_slice + run
                return run + hist3[pl.ds(l * 256 + g * 16, 16)]

            lax.fori_loop(0, 16, linit, z16)
            return 0

        lax.fori_loop(0, 16, cinit, 0)

        def pbody(i, _):
            idx = lane * CL + i
            k = plsc.load_gather(keych, [idx])
            d = lax.shift_right_logical(k, shift) & 255
            fl = lane * 256 + d
            c = plsc.load_gather(counts3, [fl])
            plsc.addupdate_scatter(counts3, [fl], ones16)
            plsc.store_scatter(posbuf, [idx], c)
            return 0

        lax.fori_loop(0, CL, pbody, 0)
        pltpu.sync_copy(keych, nxt_k.at[posbuf])
        pltpu.sync_copy(valch, nxt_v.at[posbuf])
        plsc.subcore_barrier()

    fin_v = bufs[NPASS % 2][1]
    pltpu.sync_copy(fin_v.at[pl.ds(base, CH)], valch)
    pltpu.sync_copy(ev_hbm.at[pl.ds(base, CH)], evch)

    def cumb(t, carry):
        h = jnp.exp(valch[pl.ds(t * 16, 16)])
        riskloc[pl.ds(t * 16, 16)] = plsc.cumsum(h) + carry
        return carry + jnp.sum(h)

    tot = lax.fori_loop(0, CL, cumb, jnp.float32(0.0))
    vbuf[...] = jnp.zeros((16,), jnp.float32) + tot
    pltpu.sync_copy(vbuf, stot.at[pl.ds(wid * 16, 16)])
    plsc.subcore_barrier()
    pltpu.sync_copy(stot, redloc)

    def prb(w2, acc):
        return acc + redloc[pl.ds(w2 * 16, 16)] * (w2 < wid).astype(jnp.float32)

    prevec = lax.fori_loop(0, NW, prb, jnp.zeros((16,), jnp.float32))

    def fbody(t, accs):
        am, ac = accs
        risk = riskloc[pl.ds(t * 16, 16)] + prevec
        lr = _vlog(risk + 1e-8)
        e = evch[pl.ds(t * 16, 16)].astype(jnp.float32)
        return (am + e * (valch[pl.ds(t * 16, 16)] - lr), ac + e)

    am, ac = lax.fori_loop(0, CL, fbody,
                           (jnp.zeros((16,), jnp.float32),
                            jnp.zeros((16,), jnp.float32)))
    vbuf[...] = am
    pltpu.sync_copy(vbuf, spm.at[pl.ds(wid * 16, 16)])
    vbuf[...] = ac
    pltpu.sync_copy(vbuf, spc.at[pl.ds(wid * 16, 16)])
    plsc.subcore_barrier()

    @pl.when(wid == 0)
    def _():
        pltpu.sync_copy(spm, redloc)

        def rsum(w2, acc):
            return acc + redloc[pl.ds(w2 * 16, 16)]

        zf = jnp.zeros((16,), jnp.float32)
        msv = zf + jnp.sum(lax.fori_loop(0, NW, rsum, zf))
        pltpu.sync_copy(spc, redloc)
        cntv = zf + jnp.sum(lax.fori_loop(0, NW, rsum, zf))
        vbuf[...] = jnp.where(cntv == 0.0, zf,
                              -(msv / jnp.maximum(cntv, 1.0)))
        pltpu.sync_copy(vbuf, out_hbm)


@jax.jit
def _cox_sc(durations, cox_out, ev32):
    mesh = plsc.VectorSubcoreMesh(core_axis_name="c", subcore_axis_name="s",
                                  num_cores=1)
    f = pl.kernel(
        _cox_sc_body,
        out_type=jax.ShapeDtypeStruct((16,), jnp.float32),
        mesh=mesh,
        compiler_params=pltpu.CompilerParams(needs_layout_passes=False),
        scratch_types=[
            pltpu.VMEM_SHARED((B,), jnp.int32),
            pltpu.VMEM_SHARED((B,), jnp.int32),
            pltpu.VMEM_SHARED((B,), jnp.float32),
            pltpu.VMEM_SHARED((B,), jnp.float32),
            pltpu.VMEM_SHARED((NW * 256,), jnp.int32),
            pltpu.VMEM_SHARED((NW * 16,), jnp.float32),
            pltpu.VMEM_SHARED((NW * 16,), jnp.float32),
            pltpu.VMEM_SHARED((NW * 16,), jnp.float32),
            pltpu.VMEM((CH,), jnp.int32),
            pltpu.VMEM((CH,), jnp.float32),
            pltpu.VMEM((CH,), jnp.int32),
            pltpu.VMEM((16 * 256,), jnp.int32),
            pltpu.VMEM((16 * 256,), jnp.int32),
            pltpu.VMEM((NW * 256,), jnp.int32),
            pltpu.VMEM((256,), jnp.int32),
            pltpu.VMEM((256,), jnp.int32),
            pltpu.VMEM((CH,), jnp.float32),
            pltpu.VMEM((CH,), jnp.int32),
            pltpu.VMEM((NW * 16,), jnp.float32),
            pltpu.VMEM((16,), jnp.float32),
        ],
    )
    return f(durations, cox_out, ev32)


def kernel(cls_out, y_cls, cox_out, durations, events):
    y_row = y_cls.astype(jnp.int32).reshape(1, B)
    ce_sum = pl.pallas_call(
        _ce_kernel,
        grid=(B // CB,),
        in_specs=[
            pl.BlockSpec((C, CB), lambda i: (0, i)),
            pl.BlockSpec((1, CB), lambda i: (0, i)),
        ],
        out_specs=pl.BlockSpec((1, 1), lambda i: (0, 0)),
        out_shape=jax.ShapeDtypeStruct((1, 1), jnp.float32),
    )(cls_out.T, y_row)

    cox_vec = _cox_sc(durations, cox_out, events.astype(jnp.int32))

    loss_cls = ce_sum[0, 0] / jnp.float32(B)
    loss_cox = cox_vec[0]
    total = ALPHA * loss_cls + (1.0 - ALPHA) * loss_cox
    return (total, loss_cls, loss_cox)
